# baseline (speedup 1.0000x reference)
import jax
import jax.numpy as jnp
from jax import lax
from jax.experimental import pallas as pl
from jax.experimental.pallas import tpu as pltpu
from jax.experimental.pallas import tpu_sc as plsc

NUM_CLASSES = 100000
FEAT_DIM = 64
BATCH = 16384
ALPHA = 0.5

_NTILES = 16
_PER_TILE = BATCH // _NTILES
_CW = 128
_NCH = _PER_TILE // _CW
_NGRP = _CW // 16
_CNTW = 16
_NCHC = _NCH // 2


def _sqrt16(v):
    b = lax.bitcast_convert_type(v, jnp.int32)
    y = lax.bitcast_convert_type(jnp.int32(0x5F3759DF) - (b >> 1),
                                 jnp.float32)
    h = v * 0.5
    y = y * (1.5 - (h * y) * y)
    y = y * (1.5 - (h * y) * y)
    y = y * (1.5 - (h * y) * y)
    return v * y


def _sc_body(x_hbm, lbl_hbm, out_hbm,
             tr_sh, s_sh, c_sh, p_sh,
             lbl_v, idr_v, xb0, xb1, mb0, mb1, cb, pb, ones_v, accb, redb,
             sem_lbl, sem_zero, sem_id, sem_r, sem_x, sem_sas, sem_sac,
             sem_m, sem_c, sem_out):
    wid = lax.axis_index("s")
    cid = lax.axis_index("c")
    row0 = wid * _NCH
    samp0 = wid * _PER_TILE
    sampc = (cid * _NTILES + wid) * (_PER_TILE // 2)
    rowc = (cid * _NTILES + wid) * _NCHC
    xb = (xb0, xb1)
    mb = (mb0, mb1)

    zeros16 = jnp.zeros((16,), jnp.float32)
    ones16 = jnp.ones((16,), jnp.float32)
    iota16 = lax.iota(jnp.int32, 16)

    w_lbl = pltpu.async_copy(lbl_hbm.at[pl.ds(row0, _NCH)], lbl_v, sem_lbl)

    for j in range(_NCH):
        base = (row0 + j) * _CW
        for g in range(_NGRP):
            idr_v[j, pl.ds(g * 16, 16)] = iota16 + (base + g * 16)

    @plsc.parallel_loop(0, _CW, unroll=2)
    def _zx(i):
        xb0[i, pl.ds(0, 16)] = zeros16
        xb0[i, pl.ds(16, 16)] = zeros16
        xb0[i, pl.ds(32, 16)] = zeros16
        xb0[i, pl.ds(48, 16)] = zeros16
        pb[i, pl.ds(0, _CNTW)] = zeros16
        ones_v[i, pl.ds(0, _CNTW)] = ones16

    wz = []
    for j in range(_NCH):
        wz.append(pltpu.async_copy(
            xb0, s_sh.at[pl.ds(samp0 + j * _CW, _CW)], sem_zero))
        wz.append(pltpu.async_copy(
            pb, c_sh.at[pl.ds(samp0 + j * _CW, _CW)], sem_zero))

    w_lbl.wait()
    w_id = [pltpu.async_copy(idr_v.at[j], tr_sh.at[lbl_v.at[j]], sem_id)
            for j in range(_NCH)]
    for w in wz:
        w.wait()
    for w in w_id:
        w.wait()

    plsc.subcore_barrier()

    w_r = [pltpu.async_copy(tr_sh.at[lbl_v.at[j]], idr_v.at[j], sem_r)
           for j in range(_NCH)]
    w_x = {}
    for j in range(2):
        w_x[j] = pltpu.async_copy(
            x_hbm.at[pl.ds(samp0 + j * _CW, _CW)], xb[j % 2], sem_x)
    for w in w_r:
        w.wait()
    w_sas, w_sac = {}, {}
    sas_waited = set()
    for j in range(_NCH):
        w_x[j].wait()
        w_sas[j] = pltpu.async_copy(xb[j % 2], s_sh.at[idr_v.at[j]],
                                    sem_sas, add=True)
        w_sac[j] = pltpu.async_copy(ones_v, c_sh.at[idr_v.at[j]],
                                    sem_sac, add=True)
        if j + 2 < _NCH:
            w_sas[j].wait()
            sas_waited.add(j)
            w_x[j + 2] = pltpu.async_copy(
                x_hbm.at[pl.ds(samp0 + (j + 2) * _CW, _CW)], xb[j % 2],
                sem_x)
    for j in range(_NCH):
        if j not in sas_waited:
            w_sas[j].wait()
        w_sac[j].wait()

    w_lbl = pltpu.async_copy(lbl_hbm.at[pl.ds(rowc, _NCHC)],
                             lbl_v.at[pl.ds(0, _NCHC)], sem_lbl)
    w_lbl.wait()
    w_rc = [pltpu.async_copy(tr_sh.at[lbl_v.at[j]], idr_v.at[j], sem_r)
            for j in range(_NCHC)]
    for j in range(2):
        w_x[j] = pltpu.async_copy(
            x_hbm.at[pl.ds(sampc + j * _CW, _CW)], xb[j % 2], sem_x)
    for w in w_rc:
        w.wait()

    plsc.subcore_barrier()

    w_m = {0: pltpu.async_copy(s_sh.at[idr_v.at[0]], mb[0], sem_m),
           1: pltpu.async_copy(s_sh.at[idr_v.at[1]], mb[1], sem_m)}
    w_c = {0: pltpu.async_copy(c_sh.at[idr_v.at[0]], cb, sem_c)}

    tile_acc = zeros16
    for j in range(_NCHC):
        w_x[j].wait()
        w_m[j].wait()
        w_c[j].wait()
        xbj, mbj = xb[j % 2], mb[j % 2]

        @plsc.parallel_loop(0, _CW, unroll=4)
        def _dist(i):
            inv = ones16 / cb[i, pl.ds(0, _CNTW)]
            d0 = xbj[i, pl.ds(0, 16)] - mbj[i, pl.ds(0, 16)] * inv
            d1 = xbj[i, pl.ds(16, 16)] - mbj[i, pl.ds(16, 16)] * inv
            d2 = xbj[i, pl.ds(32, 16)] - mbj[i, pl.ds(32, 16)] * inv
            d3 = xbj[i, pl.ds(48, 16)] - mbj[i, pl.ds(48, 16)] * inv
            pb[i, pl.ds(0, _CNTW)] = d0 * d0 + d1 * d1 + d2 * d2 + d3 * d3

        if j + 1 < _NCHC:
            w_c[j + 1] = pltpu.async_copy(c_sh.at[idr_v.at[j + 1]], cb,
                                          sem_c)

        def _grp(g, acc):
            rows = iota16 + g * 16
            vs = [plsc.load_gather(pb, [rows, jnp.full((16,), k, jnp.int32)])
                  for k in range(_CNTW)]
            while len(vs) > 1:
                vs = [vs[t] + vs[t + 1] for t in range(0, len(vs), 2)]
            return acc + _sqrt16(vs[0])
        tile_acc = lax.fori_loop(0, _NGRP, _grp, tile_acc)

        if j + 2 < _NCHC:
            w_m[j + 2] = pltpu.async_copy(s_sh.at[idr_v.at[j + 2]],
                                          mb[j % 2], sem_m)
            w_x[j + 2] = pltpu.async_copy(
                x_hbm.at[pl.ds(sampc + (j + 2) * _CW, _CW)], xb[j % 2],
                sem_x)

    accb[pl.ds(0, 16)] = tile_acc
    pltpu.async_copy(accb, p_sh.at[wid], sem_out).wait()
    plsc.subcore_barrier()

    @pl.when(wid == 0)
    def _():
        pltpu.async_copy(p_sh, redb, sem_out).wait()
        tot = zeros16
        for t in range(_NTILES):
            tot = tot + redb[t, pl.ds(0, 16)]
        total = jnp.sum(tot) * (ALPHA / BATCH)
        accb[pl.ds(0, 16)] = jnp.full((16,), total, jnp.float32)
        pltpu.async_copy(accb, out_hbm.at[cid], sem_out).wait()


def _make_sc_call():
    mesh = plsc.VectorSubcoreMesh(core_axis_name="c", subcore_axis_name="s",
                                  num_cores=2)
    return pl.kernel(
        _sc_body,
        out_type=jax.ShapeDtypeStruct((2, 16), jnp.float32),
        mesh=mesh,
        scratch_types=[
            pltpu.VMEM_SHARED((NUM_CLASSES,), jnp.int32),
            pltpu.VMEM_SHARED((BATCH, FEAT_DIM), jnp.float32),
            pltpu.VMEM_SHARED((BATCH, _CNTW), jnp.float32),
            pltpu.VMEM_SHARED((_NTILES, 16), jnp.float32),
            pltpu.VMEM((_NCH, _CW), jnp.int32),
            pltpu.VMEM((_NCH, _CW), jnp.int32),
            pltpu.VMEM((_CW, FEAT_DIM), jnp.float32),
            pltpu.VMEM((_CW, FEAT_DIM), jnp.float32),
            pltpu.VMEM((_CW, FEAT_DIM), jnp.float32),
            pltpu.VMEM((_CW, FEAT_DIM), jnp.float32),
            pltpu.VMEM((_CW, _CNTW), jnp.float32),
            pltpu.VMEM((_CW, _CNTW), jnp.float32),
            pltpu.VMEM((_CW, _CNTW), jnp.float32),
            pltpu.VMEM((16,), jnp.float32),
            pltpu.VMEM((_NTILES, 16), jnp.float32),
            pltpu.SemaphoreType.DMA,
            pltpu.SemaphoreType.DMA,
            pltpu.SemaphoreType.DMA,
            pltpu.SemaphoreType.DMA,
            pltpu.SemaphoreType.DMA,
            pltpu.SemaphoreType.DMA,
            pltpu.SemaphoreType.DMA,
            pltpu.SemaphoreType.DMA,
            pltpu.SemaphoreType.DMA,
            pltpu.SemaphoreType.DMA,
        ],
        compiler_params=pltpu.CompilerParams(use_tc_tiling_on_sc=False,
                                             needs_layout_passes=False),
        name="center_loss_sc",
    )


@jax.jit
def kernel(x, labels, centers):
    del centers
    lbl2d = labels.astype(jnp.int32).reshape(BATCH // _CW, _CW)
    out = _make_sc_call()(x, lbl2d)
    return out[0, 0] + out[1, 0]


# speedup vs baseline: 1.0335x; 1.0335x over previous
import jax
import jax.numpy as jnp
from jax import lax
from jax.experimental import pallas as pl
from jax.experimental.pallas import tpu as pltpu
from jax.experimental.pallas import tpu_sc as plsc

NUM_CLASSES = 100000
FEAT_DIM = 64
BATCH = 16384
ALPHA = 0.5

_NTILES = 16
_PER_TILE = BATCH // _NTILES
_CW = 128
_NCH = _PER_TILE // _CW
_NGRP = _CW // 16
_NCHC = _NCH // 2


def _sqrt16(v):
    b = lax.bitcast_convert_type(v, jnp.int32)
    y = lax.bitcast_convert_type(jnp.int32(0x5F3759DF) - (b >> 1),
                                 jnp.float32)
    h = v * 0.5
    y = y * (1.5 - (h * y) * y)
    y = y * (1.5 - (h * y) * y)
    y = y * (1.5 - (h * y) * y)
    return v * y


def _sc_body(x_hbm, lbl_hbm, out_hbm,
             tr_sh, s_sh, c_sh, p_sh,
             lbl_v, idr_v, xb0, xb1, xb2, mb0, mb1, cb0, cb1, inv1d, pb,
             ones_v, accb, redb,
             sem_lbl, sem_zero, sem_id, sem_r, sem_x, sem_sas, sem_sac,
             sem_m, sem_c, sem_out):
    wid = lax.axis_index("s")
    cid = lax.axis_index("c")
    row0 = wid * _NCH
    samp0 = wid * _PER_TILE
    sampc = (cid * _NTILES + wid) * (_PER_TILE // 2)
    rowc = (cid * _NTILES + wid) * _NCHC
    xb = (xb0, xb1, xb2)
    mb = (mb0, mb1)
    cb = (cb0, cb1)

    zeros16 = jnp.zeros((16,), jnp.float32)
    ones16 = jnp.ones((16,), jnp.float32)
    iota16 = lax.iota(jnp.int32, 16)

    w_lbl = pltpu.async_copy(lbl_hbm.at[pl.ds(row0, _NCH)], lbl_v, sem_lbl)

    for j in range(_NCH):
        base = (row0 + j) * _CW
        for g in range(_NGRP):
            idr_v[j, pl.ds(g * 16, 16)] = iota16 + (base + g * 16)

    @plsc.parallel_loop(0, _CW, unroll=2)
    def _zx(i):
        xb0[i, pl.ds(0, 16)] = zeros16
        xb0[i, pl.ds(16, 16)] = zeros16
        xb0[i, pl.ds(32, 16)] = zeros16
        xb0[i, pl.ds(48, 16)] = zeros16
    for g in range(_NGRP):
        cb0[pl.ds(g * 16, 16)] = zeros16
        ones_v[pl.ds(g * 16, 16)] = ones16

    wz = []
    for j in range(_NCH):
        wz.append(pltpu.async_copy(
            xb0, s_sh.at[pl.ds(samp0 + j * _CW, _CW)], sem_zero))
        wz.append(pltpu.async_copy(
            cb0, c_sh.at[pl.ds(samp0 + j * _CW, _CW)], sem_zero))

    w_lbl.wait()
    w_id = [pltpu.async_copy(idr_v.at[j], tr_sh.at[lbl_v.at[j]], sem_id)
            for j in range(_NCH)]
    for w in wz:
        w.wait()
    for w in w_id:
        w.wait()

    plsc.subcore_barrier()

    w_r = [pltpu.async_copy(tr_sh.at[lbl_v.at[j]], idr_v.at[j], sem_r)
           for j in range(_NCH)]
    w_x = {}
    for j in range(2):
        w_x[j] = pltpu.async_copy(
            x_hbm.at[pl.ds(samp0 + j * _CW, _CW)], xb[j % 3], sem_x)
    for w in w_r:
        w.wait()
    w_sas, w_sac = {}, {}
    sas_waited = set()
    for j in range(_NCH):
        w_x[j].wait()
        w_sas[j] = pltpu.async_copy(xb[j % 3], s_sh.at[idr_v.at[j]],
                                    sem_sas, add=True)
        w_sac[j] = pltpu.async_copy(ones_v, c_sh.at[idr_v.at[j]],
                                    sem_sac, add=True)
        if j + 2 < _NCH:
            if j >= 1:
                w_sas[j - 1].wait()
                sas_waited.add(j - 1)
            w_x[j + 2] = pltpu.async_copy(
                x_hbm.at[pl.ds(samp0 + (j + 2) * _CW, _CW)], xb[(j + 2) % 3],
                sem_x)
    for j in range(_NCH):
        if j not in sas_waited:
            w_sas[j].wait()
        w_sac[j].wait()

    w_lbl = pltpu.async_copy(lbl_hbm.at[pl.ds(rowc, _NCHC)],
                             lbl_v.at[pl.ds(0, _NCHC)], sem_lbl)
    w_lbl.wait()
    w_rc = [pltpu.async_copy(tr_sh.at[lbl_v.at[j]], idr_v.at[j], sem_r)
            for j in range(_NCHC)]
    for j in range(2):
        w_x[j] = pltpu.async_copy(
            x_hbm.at[pl.ds(sampc + j * _CW, _CW)], xb[j % 3], sem_x)
    for w in w_rc:
        w.wait()

    plsc.subcore_barrier()

    w_m = {0: pltpu.async_copy(s_sh.at[idr_v.at[0]], mb[0], sem_m),
           1: pltpu.async_copy(s_sh.at[idr_v.at[1]], mb[1], sem_m)}
    w_c = {0: pltpu.async_copy(c_sh.at[idr_v.at[0]], cb[0], sem_c),
           1: pltpu.async_copy(c_sh.at[idr_v.at[1]], cb[1], sem_c)}

    tile_acc = zeros16
    for j in range(_NCHC):
        w_x[j].wait()
        w_m[j].wait()
        w_c[j].wait()
        xbj, mbj, cbj = xb[j % 3], mb[j % 2], cb[j % 2]

        for g in range(_NGRP):
            inv1d[pl.ds(g * 16, 16)] = ones16 / cbj[pl.ds(g * 16, 16)]

        @plsc.parallel_loop(0, _CW, unroll=4)
        def _dist(i):
            inv = plsc.load_gather(inv1d, [jnp.full((16,), i, jnp.int32)])
            d0 = xbj[i, pl.ds(0, 16)] - mbj[i, pl.ds(0, 16)] * inv
            d1 = xbj[i, pl.ds(16, 16)] - mbj[i, pl.ds(16, 16)] * inv
            d2 = xbj[i, pl.ds(32, 16)] - mbj[i, pl.ds(32, 16)] * inv
            d3 = xbj[i, pl.ds(48, 16)] - mbj[i, pl.ds(48, 16)] * inv
            pb[i, pl.ds(0, 16)] = d0 * d0 + d1 * d1 + d2 * d2 + d3 * d3

        if j + 2 < _NCHC:
            w_c[j + 2] = pltpu.async_copy(c_sh.at[idr_v.at[j + 2]],
                                          cb[j % 2], sem_c)

        def _grp(g, acc):
            rows = iota16 + g * 16
            vs = [plsc.load_gather(pb, [rows, jnp.full((16,), k, jnp.int32)])
                  for k in range(16)]
            while len(vs) > 1:
                vs = [vs[t] + vs[t + 1] for t in range(0, len(vs), 2)]
            return acc + _sqrt16(vs[0])
        tile_acc = lax.fori_loop(0, _NGRP, _grp, tile_acc)

        if j + 2 < _NCHC:
            w_m[j + 2] = pltpu.async_copy(s_sh.at[idr_v.at[j + 2]],
                                          mb[j % 2], sem_m)
            w_x[j + 2] = pltpu.async_copy(
                x_hbm.at[pl.ds(sampc + (j + 2) * _CW, _CW)], xb[j % 3],
                sem_x)

    accb[pl.ds(0, 16)] = tile_acc
    pltpu.async_copy(accb, p_sh.at[wid], sem_out).wait()
    plsc.subcore_barrier()

    @pl.when(wid == 0)
    def _():
        pltpu.async_copy(p_sh, redb, sem_out).wait()
        tot = zeros16
        for t in range(_NTILES):
            tot = tot + redb[t, pl.ds(0, 16)]
        total = jnp.sum(tot) * (ALPHA / BATCH)
        accb[pl.ds(0, 16)] = jnp.full((16,), total, jnp.float32)
        pltpu.async_copy(accb, out_hbm.at[cid], sem_out).wait()


def _make_sc_call():
    mesh = plsc.VectorSubcoreMesh(core_axis_name="c", subcore_axis_name="s",
                                  num_cores=2)
    return pl.kernel(
        _sc_body,
        out_type=jax.ShapeDtypeStruct((2, 16), jnp.float32),
        mesh=mesh,
        scratch_types=[
            pltpu.VMEM_SHARED((NUM_CLASSES,), jnp.int32),
            pltpu.VMEM_SHARED((BATCH, FEAT_DIM), jnp.float32),
            pltpu.VMEM_SHARED((BATCH,), jnp.float32),
            pltpu.VMEM_SHARED((_NTILES, 16), jnp.float32),
            pltpu.VMEM((_NCH, _CW), jnp.int32),
            pltpu.VMEM((_NCH, _CW), jnp.int32),
            pltpu.VMEM((_CW, FEAT_DIM), jnp.float32),
            pltpu.VMEM((_CW, FEAT_DIM), jnp.float32),
            pltpu.VMEM((_CW, FEAT_DIM), jnp.float32),
            pltpu.VMEM((_CW, FEAT_DIM), jnp.float32),
            pltpu.VMEM((_CW, FEAT_DIM), jnp.float32),
            pltpu.VMEM((_CW,), jnp.float32),
            pltpu.VMEM((_CW,), jnp.float32),
            pltpu.VMEM((_CW,), jnp.float32),
            pltpu.VMEM((_CW, 16), jnp.float32),
            pltpu.VMEM((_CW,), jnp.float32),
            pltpu.VMEM((16,), jnp.float32),
            pltpu.VMEM((_NTILES, 16), jnp.float32),
            pltpu.SemaphoreType.DMA,
            pltpu.SemaphoreType.DMA,
            pltpu.SemaphoreType.DMA,
            pltpu.SemaphoreType.DMA,
            pltpu.SemaphoreType.DMA,
            pltpu.SemaphoreType.DMA,
            pltpu.SemaphoreType.DMA,
            pltpu.SemaphoreType.DMA,
            pltpu.SemaphoreType.DMA,
            pltpu.SemaphoreType.DMA,
        ],
        compiler_params=pltpu.CompilerParams(use_tc_tiling_on_sc=False,
                                             needs_layout_passes=False),
        name="center_loss_sc",
    )


@jax.jit
def kernel(x, labels, centers):
    del centers
    lbl2d = labels.astype(jnp.int32).reshape(BATCH // _CW, _CW)
    out = _make_sc_call()(x, lbl2d)
    return out[0, 0] + out[1, 0]
